# Initial kernel scaffold; baseline (speedup 1.0000x reference)
import jax
import jax.numpy as jnp
from jax.experimental import pallas as pl


def kernel(unique_node_ids, unique_messages, timestamps, memory_table, last_update, W_ih, W_hh, b_ih, b_hh):
    raise NotImplementedError("write your pallas kernel here")


# speedup vs baseline: 1.3299x; 1.3299x over previous
import functools

import jax
import jax.numpy as jnp
from jax import lax
from jax.experimental import pallas as pl
from jax.experimental.pallas import tpu as pltpu
from jax.experimental.pallas import tpu_sc as plsc

MM = 1000000
HH = 64
DD = 128
BB = 16384

NC = 2
NS = 16
NW = NC * NS
CH = 128
NCH = BB // NW // CH

_sc_mesh = plsc.VectorSubcoreMesh(core_axis_name="c", subcore_axis_name="s")
_sc_params = pltpu.CompilerParams(use_tc_tiling_on_sc=False)


@functools.partial(
    pl.kernel,
    mesh=_sc_mesh,
    compiler_params=_sc_params,
    out_type=jax.ShapeDtypeStruct((NW, NCH, CH, HH), jnp.float32),
    scratch_types=[
        pltpu.VMEM((NCH, CH), jnp.int32),
        pltpu.VMEM((NCH, CH, HH), jnp.float32),
        pltpu.SemaphoreType.DMA,
    ],
)
def _sc_gather(mem_hbm, ids_hbm, out_hbm, idx_v, rows_v, sem):
    wid = lax.axis_index("s") * NC + lax.axis_index("c")
    pltpu.sync_copy(ids_hbm.at[wid], idx_v)
    cps = [
        pltpu.async_copy(mem_hbm.at[idx_v.at[c]], rows_v.at[c], sem)
        for c in range(NCH)
    ]
    for cp in cps:
        cp.wait()
    pltpu.sync_copy(rows_v, out_hbm.at[wid])


@functools.partial(
    pl.kernel,
    mesh=_sc_mesh,
    compiler_params=_sc_params,
    out_type=(),
    scratch_types=[
        pltpu.VMEM((NCH, CH), jnp.int32),
        pltpu.VMEM((NCH, CH), jnp.int32),
        pltpu.VMEM((NCH, CH, HH), jnp.float32),
        pltpu.VMEM((NCH, CH), jnp.float32),
        pltpu.SemaphoreType.DMA,
        pltpu.SemaphoreType.DMA,
    ],
)
def _sc_scatter(ids_hbm, lo_hbm, hnew_hbm, ts_hbm, mem_ref, lu_ref,
                idx_v, lo_v, rows_v, ts_v, sem_r, sem_t):
    wid = lax.axis_index("s") * NC + lax.axis_index("c")
    pltpu.sync_copy(ids_hbm.at[wid], idx_v)
    pltpu.sync_copy(lo_hbm.at[wid], lo_v)
    g_r = [
        pltpu.async_copy(hnew_hbm.at[lo_v.at[c]], rows_v.at[c], sem_r)
        for c in range(NCH)
    ]
    g_t = [
        pltpu.async_copy(ts_hbm.at[lo_v.at[c]], ts_v.at[c], sem_t)
        for c in range(NCH)
    ]
    for cp in g_r:
        cp.wait()
    for cp in g_t:
        cp.wait()
    s_r = [
        pltpu.async_copy(rows_v.at[c], mem_ref.at[idx_v.at[c]], sem_r)
        for c in range(NCH)
    ]
    s_t = [
        pltpu.async_copy(ts_v.at[c], lu_ref.at[idx_v.at[c]], sem_t)
        for c in range(NCH)
    ]
    for cp in s_r:
        cp.wait()
    for cp in s_t:
        cp.wait()


BLK = 512
JCH = 512
NBLK = BB // BLK


def _gru_body(ids_blk_ref, ids_all_ref, x_ref, h_ref,
              wr_ref, wz_ref, wn_ref, ur_ref, uz_ref, un_ref,
              br_ref, bz_ref, bin_ref, bhn_ref,
              hnew_ref, lastocc_ref):
    dot = functools.partial(
        lax.dot_general,
        dimension_numbers=(((1,), (1,)), ((), ())),
        preferred_element_type=jnp.float32,
        precision=lax.Precision.HIGHEST,
    )
    x = x_ref[...]
    h = h_ref[...]
    r = jax.nn.sigmoid(dot(x, wr_ref[...]) + dot(h, ur_ref[...]) + br_ref[...])
    z = jax.nn.sigmoid(dot(x, wz_ref[...]) + dot(h, uz_ref[...]) + bz_ref[...])
    n = jnp.tanh(dot(x, wn_ref[...]) + bin_ref[...]
                 + r * (dot(h, un_ref[...]) + bhn_ref[...]))
    hnew_ref[...] = (1.0 - z) * n + z * h

    my = ids_blk_ref[0, 0, :]
    my_col = my[:, None]

    def jstep(c, acc):
        jv = ids_all_ref[0, pl.ds(c * JCH, JCH)]
        eq = my_col == jv[None, :]
        jpos = jax.lax.broadcasted_iota(jnp.int32, (BLK, JCH), 1) + c * JCH
        cand = jnp.where(eq, jpos, -1)
        return jnp.maximum(acc, jnp.max(cand, axis=1))

    acc0 = jnp.full((BLK,), -1, jnp.int32)
    lastocc_ref[0, 0, :] = lax.fori_loop(0, BB // JCH, jstep, acc0)


_gru_call = pl.pallas_call(
    _gru_body,
    grid=(NBLK,),
    in_specs=[
        pl.BlockSpec((1, 1, BLK), lambda i: (i, 0, 0)),
        pl.BlockSpec((1, BB), lambda i: (0, 0)),
        pl.BlockSpec((BLK, DD), lambda i: (i, 0)),
        pl.BlockSpec((BLK, HH), lambda i: (i, 0)),
        pl.BlockSpec((HH, DD), lambda i: (0, 0)),
        pl.BlockSpec((HH, DD), lambda i: (0, 0)),
        pl.BlockSpec((HH, DD), lambda i: (0, 0)),
        pl.BlockSpec((HH, HH), lambda i: (0, 0)),
        pl.BlockSpec((HH, HH), lambda i: (0, 0)),
        pl.BlockSpec((HH, HH), lambda i: (0, 0)),
        pl.BlockSpec((1, HH), lambda i: (0, 0)),
        pl.BlockSpec((1, HH), lambda i: (0, 0)),
        pl.BlockSpec((1, HH), lambda i: (0, 0)),
        pl.BlockSpec((1, HH), lambda i: (0, 0)),
    ],
    out_specs=[
        pl.BlockSpec((BLK, HH), lambda i: (i, 0)),
        pl.BlockSpec((1, 1, BLK), lambda i: (i, 0, 0)),
    ],
    out_shape=[
        jax.ShapeDtypeStruct((BB, HH), jnp.float32),
        jax.ShapeDtypeStruct((NBLK, 1, BLK), jnp.int32),
    ],
)


def kernel(unique_node_ids, unique_messages, timestamps, memory_table,
           last_update, W_ih, W_hh, b_ih, b_hh):
    ids = unique_node_ids.astype(jnp.int32)
    ids3 = ids.reshape(NW, NCH, CH)

    h_prev = _sc_gather(memory_table, ids3).reshape(BB, HH)

    h_new, last_occ = _gru_call(
        ids.reshape(NBLK, 1, BLK), ids.reshape(1, BB),
        unique_messages, h_prev,
        W_ih[0:HH], W_ih[HH:2 * HH], W_ih[2 * HH:],
        W_hh[0:HH], W_hh[HH:2 * HH], W_hh[2 * HH:],
        (b_ih[0:HH] + b_hh[0:HH]).reshape(1, HH),
        (b_ih[HH:2 * HH] + b_hh[HH:2 * HH]).reshape(1, HH),
        b_ih[2 * HH:].reshape(1, HH),
        b_hh[2 * HH:].reshape(1, HH),
    )
    lo3 = last_occ.reshape(NW, NCH, CH)

    mem_ref = jax.new_ref(memory_table)
    lu_ref = jax.new_ref(last_update)
    _sc_scatter(ids3, lo3, h_new, timestamps, mem_ref, lu_ref)

    return (mem_ref[...][:, None, :], lu_ref[...])
